# baseline (speedup 1.0000x reference)
import functools

import jax
import jax.numpy as jnp
from jax import lax
from jax.experimental import pallas as pl
from jax.experimental.pallas import tpu as pltpu
from jax.experimental.pallas import tpu_sc as plsc

G = 64
EK = 125


def _init_body(x_ref, wT_ref, b_ref, wg_ref, h_ref, m_ref):
    h = jnp.dot(x_ref[...], wT_ref[...], preferred_element_type=jnp.float32)
    h = h + b_ref[...]
    h_ref[...] = h
    m_ref[...] = jnp.dot(h, wg_ref[...], preferred_element_type=jnp.float32)


def _init_proj(x, lin_WT, lin_b2, wg0, blk):
    n, f = x.shape
    return pl.pallas_call(
        _init_body,
        grid=(n // blk,),
        in_specs=[
            pl.BlockSpec((blk, f), lambda i: (i, 0)),
            pl.BlockSpec((f, f), lambda i: (0, 0)),
            pl.BlockSpec((1, f), lambda i: (0, 0)),
            pl.BlockSpec((f, f), lambda i: (0, 0)),
        ],
        out_specs=[
            pl.BlockSpec((blk, f), lambda i: (i, 0)),
            pl.BlockSpec((blk, f), lambda i: (i, 0)),
        ],
        out_shape=[jax.ShapeDtypeStruct((n, f), jnp.float32)] * 2,
    )(x, lin_WT, lin_b2, wg0)


def _gru_body(h_ref, agg_ref, wihT_ref, whhT_ref, bih_ref, bhh_ref, wgn_ref,
              hn_ref, mn_ref):
    f = h_ref.shape[1]
    h = h_ref[...]
    agg = agg_ref[0] + agg_ref[1]
    gi = jnp.dot(agg, wihT_ref[...], preferred_element_type=jnp.float32)
    gi = gi + bih_ref[...]
    gh = jnp.dot(h, whhT_ref[...], preferred_element_type=jnp.float32)
    gh = gh + bhh_ref[...]
    r = jax.nn.sigmoid(gi[:, :f] + gh[:, :f])
    z = jax.nn.sigmoid(gi[:, f:2 * f] + gh[:, f:2 * f])
    n = jnp.tanh(gi[:, 2 * f:] + r * gh[:, 2 * f:])
    hn = (1.0 - z) * n + z * h
    hn_ref[...] = hn
    mn_ref[...] = jnp.dot(hn, wgn_ref[...], preferred_element_type=jnp.float32)


def _gru_step(h, agg2, wihT, whhT, bih2, bhh2, wg_next, blk):
    n, f = h.shape
    return pl.pallas_call(
        _gru_body,
        grid=(n // blk,),
        in_specs=[
            pl.BlockSpec((blk, f), lambda i: (i, 0)),
            pl.BlockSpec((2, blk, f), lambda i: (0, i, 0)),
            pl.BlockSpec((f, 3 * f), lambda i: (0, 0)),
            pl.BlockSpec((f, 3 * f), lambda i: (0, 0)),
            pl.BlockSpec((1, 3 * f), lambda i: (0, 0)),
            pl.BlockSpec((1, 3 * f), lambda i: (0, 0)),
            pl.BlockSpec((f, f), lambda i: (0, 0)),
        ],
        out_specs=[
            pl.BlockSpec((blk, f), lambda i: (i, 0)),
            pl.BlockSpec((blk, f), lambda i: (i, 0)),
        ],
        out_shape=[jax.ShapeDtypeStruct((n, f), jnp.float32)] * 2,
    )(h, agg2, wihT, whhT, bih2, bhh2, wg_next)


def _pool_body(h_ref, b3_ref, c1T_ref, c1b_ref, c2T_ref, c2b_ref, out_ref,
               sum_ref, cnt_ref):
    i = pl.program_id(0)
    nb = pl.num_programs(0)
    g = sum_ref.shape[0]

    @pl.when(i == 0)
    def _():
        sum_ref[...] = jnp.zeros_like(sum_ref)
        cnt_ref[...] = jnp.zeros_like(cnt_ref)

    batch = b3_ref[0, 0, :]
    gids = lax.broadcasted_iota(jnp.int32, (g, batch.shape[0]), 0)
    onehot = (batch[None, :] == gids).astype(jnp.float32)
    sum_ref[...] += jnp.dot(onehot, h_ref[...],
                            preferred_element_type=jnp.float32)
    cnt_ref[...] += jnp.sum(onehot, axis=1, keepdims=True)

    @pl.when(i == nb - 1)
    def _():
        pooled = sum_ref[...] / jnp.maximum(cnt_ref[...], 1.0)
        hid = jnp.dot(pooled, c1T_ref[...], preferred_element_type=jnp.float32)
        hid = jax.nn.relu(hid + c1b_ref[...])
        logits = jnp.dot(hid, c2T_ref[...], preferred_element_type=jnp.float32)
        out_ref[...] = jax.nn.sigmoid(logits + c2b_ref[...])


def _pool_classify(h, batch3, c1T, c1b2, c2T, c2b2, blk):
    n, f = h.shape
    nb = n // blk
    return pl.pallas_call(
        _pool_body,
        grid=(nb,),
        in_specs=[
            pl.BlockSpec((blk, f), lambda i: (i, 0)),
            pl.BlockSpec((1, 1, blk), lambda i: (i, 0, 0)),
            pl.BlockSpec((f, f), lambda i: (0, 0)),
            pl.BlockSpec((1, f), lambda i: (0, 0)),
            pl.BlockSpec((f, 1), lambda i: (0, 0)),
            pl.BlockSpec((1, 1), lambda i: (0, 0)),
        ],
        out_specs=pl.BlockSpec((G, 1), lambda i: (0, 0)),
        out_shape=jax.ShapeDtypeStruct((G, 1), jnp.float32),
        scratch_shapes=[
            pltpu.VMEM((G, f), jnp.float32),
            pltpu.VMEM((G, 1), jnp.float32),
        ],
    )(h, batch3, c1T, c1b2, c2T, c2b2)


@functools.lru_cache(maxsize=None)
def _make_edge_agg(n, f, nchunk):
    info = plsc.get_sparse_core_info()
    nc, ns = info.num_cores, info.num_subcores
    nw = nc * ns
    cpw = nchunk // nw
    rpt = (n // ns) // 8 * 8
    tail = n - rpt * ns
    mesh = plsc.VectorSubcoreMesh(core_axis_name="c", subcore_axis_name="s")

    nbuf = 2
    nph = 2
    hpc = cpw // nph
    nq = hpc // nbuf

    @functools.partial(
        pl.kernel, mesh=mesh,
        out_type=jax.ShapeDtypeStruct((nc, n, f), jnp.float32),
        scratch_types=[
            pltpu.VMEM((hpc, EK), jnp.int32),
            pltpu.VMEM((hpc, EK), jnp.int32),
        ] + [pltpu.VMEM((EK, f), jnp.float32)] * nbuf
          + [pltpu.VMEM_SHARED((n, f), jnp.float32)]
          + [pltpu.SemaphoreType.DMA] * (2 * nbuf),
    )
    def edge_agg(m_hbm, src_hbm, dst_hbm, z_hbm, out_hbm,
                 src_v, dst_v, *rest):
        bufs = rest[:nbuf]
        acc = rest[nbuf]
        gsems = rest[nbuf + 1:nbuf + 1 + nbuf]
        ssems = rest[nbuf + 1 + nbuf:]
        c = lax.axis_index("c")
        s = lax.axis_index("s")
        wid = s * nc + c
        base_r = s * rpt
        pltpu.sync_copy(z_hbm.at[pl.ds(base_r, rpt)], acc.at[pl.ds(base_r, rpt)])
        if tail:
            @pl.when(s == ns - 1)
            def _():
                pltpu.sync_copy(z_hbm.at[pl.ds(rpt * ns, tail)],
                                acc.at[pl.ds(rpt * ns, tail)])
        plsc.subcore_barrier()

        def gather(j, b):
            pltpu.async_copy(m_hbm.at[src_v.at[j]], bufs[b], gsems[b])

        def gather_wait(j, b):
            pltpu.make_async_copy(m_hbm.at[src_v.at[j]], bufs[b],
                                  gsems[b]).wait()

        def scat(j, b):
            pltpu.async_copy(bufs[b], acc.at[dst_v.at[j]], ssems[b],
                             add=True)

        def scat_wait(j, b):
            pltpu.make_async_copy(bufs[b], acc.at[dst_v.at[j]],
                                  ssems[b]).wait()

        def body(q, carry, last):
            j0 = q * nbuf
            for b in range(nbuf):
                gather_wait(j0 + b, b)
                scat(j0 + b, b)
            for b in range(nbuf):
                scat_wait(j0 + b, b)
                if not last:
                    gather(j0 + nbuf + b, b)
            return carry

        for ph in range(nph):
            cbase = wid * cpw + ph * hpc
            pltpu.sync_copy(src_hbm.at[pl.ds(cbase, hpc)], src_v)
            pltpu.sync_copy(dst_hbm.at[pl.ds(cbase, hpc)], dst_v)
            for b in range(nbuf):
                gather(b, b)
            lax.fori_loop(0, nq - 1, lambda q, cy: body(q, cy, False), 0)
            body(nq - 1, 0, True)
        plsc.subcore_barrier()
        pltpu.sync_copy(acc.at[pl.ds(base_r, rpt)],
                        out_hbm.at[c].at[pl.ds(base_r, rpt)])
        if tail:
            @pl.when(s == ns - 1)
            def _():
                pltpu.sync_copy(acc.at[pl.ds(rpt * ns, tail)],
                                out_hbm.at[c].at[pl.ds(rpt * ns, tail)])

    return edge_agg


def kernel(x, edge_index, batch, lin_W, lin_b, ggnn_W, gru_W_ih, gru_W_hh,
           gru_b_ih, gru_b_hh, c1_W, c1_b, c2_W, c2_b):
    n, f = x.shape
    num_layers = ggnn_W.shape[0]
    e = edge_index.shape[1]
    blk = 2000

    src2d = edge_index[0].reshape(e // EK, EK)
    dst2d = edge_index[1].reshape(e // EK, EK)
    zeros = jnp.zeros((n, f), jnp.float32)
    lin_WT = lin_W.T
    lin_b2 = lin_b.reshape(1, f)
    wihT = gru_W_ih.T
    whhT = gru_W_hh.T
    bih2 = gru_b_ih.reshape(1, 3 * f)
    bhh2 = gru_b_hh.reshape(1, 3 * f)
    c1T = c1_W.T
    c1b2 = c1_b.reshape(1, f)
    c2T = c2_W.T
    c2b2 = c2_b.reshape(1, 1)
    batch3 = batch.reshape(n // blk, 1, blk)

    edge_agg = _make_edge_agg(n, f, e // EK)

    h, m = _init_proj(x, lin_WT, lin_b2, ggnn_W[0], blk)
    for i in range(num_layers):
        agg2 = edge_agg(m, src2d, dst2d, zeros)
        wg_next = ggnn_W[(i + 1) % num_layers]
        h, m = _gru_step(h, agg2, wihT, whhT, bih2, bhh2, wg_next, blk)
    return _pool_classify(h, batch3, c1T, c1b2, c2T, c2b2, blk)


# speedup vs baseline: 1.2860x; 1.2860x over previous
import functools

import jax
import jax.numpy as jnp
from jax import lax
from jax.experimental import pallas as pl
from jax.experimental.pallas import tpu as pltpu
from jax.experimental.pallas import tpu_sc as plsc

G = 64
EK = 125


def _init_body(x_ref, wT_ref, b_ref, wg_ref, h_ref, m_ref):
    h = jnp.dot(x_ref[...], wT_ref[...], preferred_element_type=jnp.float32)
    h = h + b_ref[...]
    h_ref[...] = h
    m = jnp.dot(h, wg_ref[...], preferred_element_type=jnp.float32)
    m_ref[...] = m.astype(m_ref.dtype)


def _init_proj(x, lin_WT, lin_b2, wg0, blk):
    n, f = x.shape
    return pl.pallas_call(
        _init_body,
        grid=(n // blk,),
        in_specs=[
            pl.BlockSpec((blk, f), lambda i: (i, 0)),
            pl.BlockSpec((f, f), lambda i: (0, 0)),
            pl.BlockSpec((1, f), lambda i: (0, 0)),
            pl.BlockSpec((f, f), lambda i: (0, 0)),
        ],
        out_specs=[
            pl.BlockSpec((blk, f), lambda i: (i, 0)),
            pl.BlockSpec((blk, f), lambda i: (i, 0)),
        ],
        out_shape=[jax.ShapeDtypeStruct((n, f), jnp.float32)] * 2,
    )(x, lin_WT, lin_b2, wg0)


def _gru_body(h_ref, agg_ref, wihT_ref, whhT_ref, bih_ref, bhh_ref, wgn_ref,
              hn_ref, mn_ref):
    f = h_ref.shape[1]
    h = h_ref[...]
    agg = agg_ref[0] + agg_ref[1]
    gi = jnp.dot(agg, wihT_ref[...], preferred_element_type=jnp.float32)
    gi = gi + bih_ref[...]
    gh = jnp.dot(h, whhT_ref[...], preferred_element_type=jnp.float32)
    gh = gh + bhh_ref[...]
    r = jax.nn.sigmoid(gi[:, :f] + gh[:, :f])
    z = jax.nn.sigmoid(gi[:, f:2 * f] + gh[:, f:2 * f])
    n = jnp.tanh(gi[:, 2 * f:] + r * gh[:, 2 * f:])
    hn = (1.0 - z) * n + z * h
    hn_ref[...] = hn
    mn = jnp.dot(hn, wgn_ref[...], preferred_element_type=jnp.float32)
    mn_ref[...] = mn.astype(mn_ref.dtype)


def _gru_step(h, agg2, wihT, whhT, bih2, bhh2, wg_next, blk):
    n, f = h.shape
    return pl.pallas_call(
        _gru_body,
        grid=(n // blk,),
        in_specs=[
            pl.BlockSpec((blk, f), lambda i: (i, 0)),
            pl.BlockSpec((2, blk, f), lambda i: (0, i, 0)),
            pl.BlockSpec((f, 3 * f), lambda i: (0, 0)),
            pl.BlockSpec((f, 3 * f), lambda i: (0, 0)),
            pl.BlockSpec((1, 3 * f), lambda i: (0, 0)),
            pl.BlockSpec((1, 3 * f), lambda i: (0, 0)),
            pl.BlockSpec((f, f), lambda i: (0, 0)),
        ],
        out_specs=[
            pl.BlockSpec((blk, f), lambda i: (i, 0)),
            pl.BlockSpec((blk, f), lambda i: (i, 0)),
        ],
        out_shape=[jax.ShapeDtypeStruct((n, f), jnp.float32)] * 2,
    )(h, agg2, wihT, whhT, bih2, bhh2, wg_next)


def _pool_body(h_ref, b3_ref, c1T_ref, c1b_ref, c2T_ref, c2b_ref, out_ref,
               sum_ref, cnt_ref):
    i = pl.program_id(0)
    nb = pl.num_programs(0)
    g = sum_ref.shape[0]

    @pl.when(i == 0)
    def _():
        sum_ref[...] = jnp.zeros_like(sum_ref)
        cnt_ref[...] = jnp.zeros_like(cnt_ref)

    batch = b3_ref[0, 0, :]
    gids = lax.broadcasted_iota(jnp.int32, (g, batch.shape[0]), 0)
    onehot = (batch[None, :] == gids).astype(jnp.float32)
    sum_ref[...] += jnp.dot(onehot, h_ref[...],
                            preferred_element_type=jnp.float32)
    cnt_ref[...] += jnp.sum(onehot, axis=1, keepdims=True)

    @pl.when(i == nb - 1)
    def _():
        pooled = sum_ref[...] / jnp.maximum(cnt_ref[...], 1.0)
        hid = jnp.dot(pooled, c1T_ref[...], preferred_element_type=jnp.float32)
        hid = jax.nn.relu(hid + c1b_ref[...])
        logits = jnp.dot(hid, c2T_ref[...], preferred_element_type=jnp.float32)
        out_ref[...] = jax.nn.sigmoid(logits + c2b_ref[...])


def _pool_classify(h, batch3, c1T, c1b2, c2T, c2b2, blk):
    n, f = h.shape
    nb = n // blk
    return pl.pallas_call(
        _pool_body,
        grid=(nb,),
        in_specs=[
            pl.BlockSpec((blk, f), lambda i: (i, 0)),
            pl.BlockSpec((1, 1, blk), lambda i: (i, 0, 0)),
            pl.BlockSpec((f, f), lambda i: (0, 0)),
            pl.BlockSpec((1, f), lambda i: (0, 0)),
            pl.BlockSpec((f, 1), lambda i: (0, 0)),
            pl.BlockSpec((1, 1), lambda i: (0, 0)),
        ],
        out_specs=pl.BlockSpec((G, 1), lambda i: (0, 0)),
        out_shape=jax.ShapeDtypeStruct((G, 1), jnp.float32),
        scratch_shapes=[
            pltpu.VMEM((G, f), jnp.float32),
            pltpu.VMEM((G, 1), jnp.float32),
        ],
    )(h, batch3, c1T, c1b2, c2T, c2b2)


@functools.lru_cache(maxsize=None)
def _make_edge_agg(n, f, nchunk):
    info = plsc.get_sparse_core_info()
    nc, ns = info.num_cores, info.num_subcores
    nw = nc * ns
    cpw = nchunk // nw
    rpt = (n // ns) // 8 * 8
    tail = n - rpt * ns
    mesh = plsc.VectorSubcoreMesh(core_axis_name="c", subcore_axis_name="s")

    nbuf = 2
    nph = 2
    hpc = cpw // nph
    nq = hpc // nbuf

    @functools.partial(
        pl.kernel, mesh=mesh,
        out_type=jax.ShapeDtypeStruct((nc, n, f), jnp.float32),
        scratch_types=[
            pltpu.VMEM((hpc, EK), jnp.int32),
            pltpu.VMEM((hpc, EK), jnp.int32),
        ] + [pltpu.VMEM((EK, f), jnp.float32)] * nbuf
          + [pltpu.VMEM_SHARED((n, f), jnp.float32)]
          + [pltpu.SemaphoreType.DMA] * nbuf,
    )
    def edge_agg(m_hbm, src_hbm, dst_hbm, z_hbm, out_hbm,
                 src_v, dst_v, *rest):
        bufs = rest[:nbuf]
        acc = rest[nbuf]
        gsems = rest[nbuf + 1:nbuf + 1 + nbuf]
        c = lax.axis_index("c")
        s = lax.axis_index("s")
        wid = s * nc + c
        base_r = s * rpt

        def gather(j, b):
            pltpu.async_copy(m_hbm.at[src_v.at[j]], bufs[b], gsems[b])

        def gather_wait(j, b):
            pltpu.make_async_copy(m_hbm.at[src_v.at[j]], bufs[b],
                                  gsems[b]).wait()

        def scat(j, b):
            pltpu.sync_copy(bufs[b], acc.at[dst_v.at[j]], add=True)

        def body(q, carry, last):
            j0 = q * nbuf
            for b in range(nbuf):
                gather_wait(j0 + b, b)
                scat(j0 + b, b)
                if not last:
                    gather(j0 + nbuf + b, b)
            return carry

        pltpu.sync_copy(src_hbm.at[pl.ds(wid * cpw, hpc)], src_v)
        pltpu.sync_copy(dst_hbm.at[pl.ds(wid * cpw, hpc)], dst_v)
        for b in range(nbuf):
            gather(b, b)
        pltpu.sync_copy(z_hbm.at[pl.ds(base_r, rpt)], acc.at[pl.ds(base_r, rpt)])
        if tail:
            @pl.when(s == ns - 1)
            def _():
                pltpu.sync_copy(z_hbm.at[pl.ds(rpt * ns, tail)],
                                acc.at[pl.ds(rpt * ns, tail)])
        plsc.subcore_barrier()

        for ph in range(nph):
            if ph:
                cbase = wid * cpw + ph * hpc
                pltpu.sync_copy(src_hbm.at[pl.ds(cbase, hpc)], src_v)
                pltpu.sync_copy(dst_hbm.at[pl.ds(cbase, hpc)], dst_v)
                for b in range(nbuf):
                    gather(b, b)
            lax.fori_loop(0, nq - 1, lambda q, cy: body(q, cy, False), 0)
            body(nq - 1, 0, True)
        plsc.subcore_barrier()
        pltpu.sync_copy(acc.at[pl.ds(base_r, rpt)],
                        out_hbm.at[c].at[pl.ds(base_r, rpt)])
        if tail:
            @pl.when(s == ns - 1)
            def _():
                pltpu.sync_copy(acc.at[pl.ds(rpt * ns, tail)],
                                out_hbm.at[c].at[pl.ds(rpt * ns, tail)])

    return edge_agg


def kernel(x, edge_index, batch, lin_W, lin_b, ggnn_W, gru_W_ih, gru_W_hh,
           gru_b_ih, gru_b_hh, c1_W, c1_b, c2_W, c2_b):
    n, f = x.shape
    num_layers = ggnn_W.shape[0]
    e = edge_index.shape[1]
    blk = 2000

    src2d = edge_index[0].reshape(e // EK, EK)
    dst2d = edge_index[1].reshape(e // EK, EK)
    zeros = jnp.zeros((n, f), jnp.float32)
    lin_WT = lin_W.T
    lin_b2 = lin_b.reshape(1, f)
    wihT = gru_W_ih.T
    whhT = gru_W_hh.T
    bih2 = gru_b_ih.reshape(1, 3 * f)
    bhh2 = gru_b_hh.reshape(1, 3 * f)
    c1T = c1_W.T
    c1b2 = c1_b.reshape(1, f)
    c2T = c2_W.T
    c2b2 = c2_b.reshape(1, 1)
    batch3 = batch.reshape(n // blk, 1, blk)

    edge_agg = _make_edge_agg(n, f, e // EK)

    h, m = _init_proj(x, lin_WT, lin_b2, ggnn_W[0], blk)
    for i in range(num_layers):
        agg2 = edge_agg(m, src2d, dst2d, zeros)
        wg_next = ggnn_W[(i + 1) % num_layers]
        h, m = _gru_step(h, agg2, wihT, whhT, bih2, bhh2, wg_next, blk)
    return _pool_classify(h, batch3, c1T, c1b2, c2T, c2b2, blk)


# speedup vs baseline: 1.2976x; 1.0091x over previous
import functools

import jax
import jax.numpy as jnp
from jax import lax
from jax.experimental import pallas as pl
from jax.experimental.pallas import tpu as pltpu
from jax.experimental.pallas import tpu_sc as plsc

G = 64
EK = 125


def _init_body(x_ref, wT_ref, b_ref, wg_ref, h_ref, m_ref):
    h = jnp.dot(x_ref[...], wT_ref[...], preferred_element_type=jnp.float32)
    h = h + b_ref[...]
    h_ref[...] = h
    m = jnp.dot(h, wg_ref[...], preferred_element_type=jnp.float32)
    m_ref[...] = m.astype(m_ref.dtype)


def _init_proj(x, lin_WT, lin_b2, wg0, blk):
    n, f = x.shape
    return pl.pallas_call(
        _init_body,
        grid=(n // blk,),
        in_specs=[
            pl.BlockSpec((blk, f), lambda i: (i, 0)),
            pl.BlockSpec((f, f), lambda i: (0, 0)),
            pl.BlockSpec((1, f), lambda i: (0, 0)),
            pl.BlockSpec((f, f), lambda i: (0, 0)),
        ],
        out_specs=[
            pl.BlockSpec((blk, f), lambda i: (i, 0)),
            pl.BlockSpec((blk, f), lambda i: (i, 0)),
        ],
        out_shape=[jax.ShapeDtypeStruct((n, f), jnp.float32)] * 2,
    )(x, lin_WT, lin_b2, wg0)


def _gru_body(h_ref, agg_ref, wihT_ref, whhT_ref, bih_ref, bhh_ref, wgn_ref,
              hn_ref, mn_ref):
    f = h_ref.shape[1]
    h = h_ref[...]
    agg = agg_ref[0] + agg_ref[1]
    gi = jnp.dot(agg, wihT_ref[...], preferred_element_type=jnp.float32)
    gi = gi + bih_ref[...]
    gh = jnp.dot(h, whhT_ref[...], preferred_element_type=jnp.float32)
    gh = gh + bhh_ref[...]
    r = jax.nn.sigmoid(gi[:, :f] + gh[:, :f])
    z = jax.nn.sigmoid(gi[:, f:2 * f] + gh[:, f:2 * f])
    n = jnp.tanh(gi[:, 2 * f:] + r * gh[:, 2 * f:])
    hn = (1.0 - z) * n + z * h
    hn_ref[...] = hn
    mn = jnp.dot(hn, wgn_ref[...], preferred_element_type=jnp.float32)
    mn_ref[...] = mn.astype(mn_ref.dtype)


def _gru_step(h, agg2, wihT, whhT, bih2, bhh2, wg_next, blk):
    n, f = h.shape
    return pl.pallas_call(
        _gru_body,
        grid=(n // blk,),
        in_specs=[
            pl.BlockSpec((blk, f), lambda i: (i, 0)),
            pl.BlockSpec((2, blk, f), lambda i: (0, i, 0)),
            pl.BlockSpec((f, 3 * f), lambda i: (0, 0)),
            pl.BlockSpec((f, 3 * f), lambda i: (0, 0)),
            pl.BlockSpec((1, 3 * f), lambda i: (0, 0)),
            pl.BlockSpec((1, 3 * f), lambda i: (0, 0)),
            pl.BlockSpec((f, f), lambda i: (0, 0)),
        ],
        out_specs=[
            pl.BlockSpec((blk, f), lambda i: (i, 0)),
            pl.BlockSpec((blk, f), lambda i: (i, 0)),
        ],
        out_shape=[jax.ShapeDtypeStruct((n, f), jnp.float32)] * 2,
    )(h, agg2, wihT, whhT, bih2, bhh2, wg_next)


def _final_body(h_ref, agg_ref, wihT_ref, whhT_ref, bih_ref, bhh_ref,
                b3_ref, c1T_ref, c1b_ref, c2T_ref, c2b_ref, out_ref,
                sum_ref, cnt_ref):
    i = pl.program_id(0)
    nb = pl.num_programs(0)
    g = sum_ref.shape[0]
    f = h_ref.shape[1]

    @pl.when(i == 0)
    def _():
        sum_ref[...] = jnp.zeros_like(sum_ref)
        cnt_ref[...] = jnp.zeros_like(cnt_ref)

    h = h_ref[...]
    agg = agg_ref[0] + agg_ref[1]
    gi = jnp.dot(agg, wihT_ref[...], preferred_element_type=jnp.float32)
    gi = gi + bih_ref[...]
    gh = jnp.dot(h, whhT_ref[...], preferred_element_type=jnp.float32)
    gh = gh + bhh_ref[...]
    r = jax.nn.sigmoid(gi[:, :f] + gh[:, :f])
    z = jax.nn.sigmoid(gi[:, f:2 * f] + gh[:, f:2 * f])
    n = jnp.tanh(gi[:, 2 * f:] + r * gh[:, 2 * f:])
    hn = (1.0 - z) * n + z * h

    batch = b3_ref[0, 0, :]
    gids = lax.broadcasted_iota(jnp.int32, (g, batch.shape[0]), 0)
    onehot = (batch[None, :] == gids).astype(jnp.float32)
    sum_ref[...] += jnp.dot(onehot, hn, preferred_element_type=jnp.float32)
    cnt_ref[...] += jnp.sum(onehot, axis=1, keepdims=True)

    @pl.when(i == nb - 1)
    def _():
        pooled = sum_ref[...] / jnp.maximum(cnt_ref[...], 1.0)
        hid = jnp.dot(pooled, c1T_ref[...], preferred_element_type=jnp.float32)
        hid = jax.nn.relu(hid + c1b_ref[...])
        logits = jnp.dot(hid, c2T_ref[...], preferred_element_type=jnp.float32)
        out_ref[...] = jax.nn.sigmoid(logits + c2b_ref[...])


def _final_step(h, agg2, wihT, whhT, bih2, bhh2, batch3, c1T, c1b2, c2T,
                c2b2, blk):
    n, f = h.shape
    nb = n // blk
    return pl.pallas_call(
        _final_body,
        grid=(nb,),
        in_specs=[
            pl.BlockSpec((blk, f), lambda i: (i, 0)),
            pl.BlockSpec((2, blk, f), lambda i: (0, i, 0)),
            pl.BlockSpec((f, 3 * f), lambda i: (0, 0)),
            pl.BlockSpec((f, 3 * f), lambda i: (0, 0)),
            pl.BlockSpec((1, 3 * f), lambda i: (0, 0)),
            pl.BlockSpec((1, 3 * f), lambda i: (0, 0)),
            pl.BlockSpec((1, 1, blk), lambda i: (i, 0, 0)),
            pl.BlockSpec((f, f), lambda i: (0, 0)),
            pl.BlockSpec((1, f), lambda i: (0, 0)),
            pl.BlockSpec((f, 1), lambda i: (0, 0)),
            pl.BlockSpec((1, 1), lambda i: (0, 0)),
        ],
        out_specs=pl.BlockSpec((G, 1), lambda i: (0, 0)),
        out_shape=jax.ShapeDtypeStruct((G, 1), jnp.float32),
        scratch_shapes=[
            pltpu.VMEM((G, f), jnp.float32),
            pltpu.VMEM((G, 1), jnp.float32),
        ],
    )(h, agg2, wihT, whhT, bih2, bhh2, batch3, c1T, c1b2, c2T, c2b2)


@functools.lru_cache(maxsize=None)
def _make_edge_agg(n, f, nchunk):
    info = plsc.get_sparse_core_info()
    nc, ns = info.num_cores, info.num_subcores
    nw = nc * ns
    cpw = nchunk // nw
    rpt = (n // ns) // 8 * 8
    tail = n - rpt * ns
    mesh = plsc.VectorSubcoreMesh(core_axis_name="c", subcore_axis_name="s")

    nbuf = 2
    nph = 2
    hpc = cpw // nph
    nq = hpc // nbuf

    @functools.partial(
        pl.kernel, mesh=mesh,
        out_type=jax.ShapeDtypeStruct((nc, n, f), jnp.float32),
        scratch_types=[
            pltpu.VMEM((hpc, EK), jnp.int32),
            pltpu.VMEM((hpc, EK), jnp.int32),
        ] + [pltpu.VMEM((EK, f), jnp.float32)] * nbuf
          + [pltpu.VMEM_SHARED((n, f), jnp.float32)]
          + [pltpu.SemaphoreType.DMA] * nbuf,
    )
    def edge_agg(m_hbm, src_hbm, dst_hbm, z_hbm, out_hbm,
                 src_v, dst_v, *rest):
        bufs = rest[:nbuf]
        acc = rest[nbuf]
        gsems = rest[nbuf + 1:nbuf + 1 + nbuf]
        c = lax.axis_index("c")
        s = lax.axis_index("s")
        wid = s * nc + c
        base_r = s * rpt

        def gather(j, b):
            pltpu.async_copy(m_hbm.at[src_v.at[j]], bufs[b], gsems[b])

        def gather_wait(j, b):
            pltpu.make_async_copy(m_hbm.at[src_v.at[j]], bufs[b],
                                  gsems[b]).wait()

        def scat(j, b):
            pltpu.sync_copy(bufs[b], acc.at[dst_v.at[j]], add=True)

        def body(q, carry, last):
            j0 = q * nbuf
            for b in range(nbuf):
                gather_wait(j0 + b, b)
                scat(j0 + b, b)
                if not last:
                    gather(j0 + nbuf + b, b)
            return carry

        pltpu.sync_copy(src_hbm.at[pl.ds(wid * cpw, hpc)], src_v)
        pltpu.sync_copy(dst_hbm.at[pl.ds(wid * cpw, hpc)], dst_v)
        for b in range(nbuf):
            gather(b, b)
        pltpu.sync_copy(z_hbm.at[pl.ds(base_r, rpt)], acc.at[pl.ds(base_r, rpt)])
        if tail:
            @pl.when(s == ns - 1)
            def _():
                pltpu.sync_copy(z_hbm.at[pl.ds(rpt * ns, tail)],
                                acc.at[pl.ds(rpt * ns, tail)])
        plsc.subcore_barrier()

        for ph in range(nph):
            if ph:
                cbase = wid * cpw + ph * hpc
                pltpu.sync_copy(src_hbm.at[pl.ds(cbase, hpc)], src_v)
                pltpu.sync_copy(dst_hbm.at[pl.ds(cbase, hpc)], dst_v)
                for b in range(nbuf):
                    gather(b, b)
            lax.fori_loop(0, nq - 1, lambda q, cy: body(q, cy, False), 0)
            body(nq - 1, 0, True)
        plsc.subcore_barrier()
        pltpu.sync_copy(acc.at[pl.ds(base_r, rpt)],
                        out_hbm.at[c].at[pl.ds(base_r, rpt)])
        if tail:
            @pl.when(s == ns - 1)
            def _():
                pltpu.sync_copy(acc.at[pl.ds(rpt * ns, tail)],
                                out_hbm.at[c].at[pl.ds(rpt * ns, tail)])

    return edge_agg


def kernel(x, edge_index, batch, lin_W, lin_b, ggnn_W, gru_W_ih, gru_W_hh,
           gru_b_ih, gru_b_hh, c1_W, c1_b, c2_W, c2_b):
    n, f = x.shape
    num_layers = ggnn_W.shape[0]
    e = edge_index.shape[1]
    blk = 2000

    src2d = edge_index[0].reshape(e // EK, EK)
    dst2d = edge_index[1].reshape(e // EK, EK)
    zeros = jnp.zeros((n, f), jnp.float32)
    lin_WT = lin_W.T
    lin_b2 = lin_b.reshape(1, f)
    wihT = gru_W_ih.T
    whhT = gru_W_hh.T
    bih2 = gru_b_ih.reshape(1, 3 * f)
    bhh2 = gru_b_hh.reshape(1, 3 * f)
    c1T = c1_W.T
    c1b2 = c1_b.reshape(1, f)
    c2T = c2_W.T
    c2b2 = c2_b.reshape(1, 1)
    batch3 = batch.reshape(n // blk, 1, blk)

    edge_agg = _make_edge_agg(n, f, e // EK)

    h, m = _init_proj(x, lin_WT, lin_b2, ggnn_W[0], blk)
    for i in range(num_layers - 1):
        agg2 = edge_agg(m, src2d, dst2d, zeros)
        h, m = _gru_step(h, agg2, wihT, whhT, bih2, bhh2, ggnn_W[i + 1], blk)
    agg2 = edge_agg(m, src2d, dst2d, zeros)
    return _final_step(h, agg2, wihT, whhT, bih2, bhh2, batch3, c1T, c1b2,
                       c2T, c2b2, blk)


# speedup vs baseline: 1.3048x; 1.0056x over previous
import functools

import jax
import jax.numpy as jnp
from jax import lax
from jax.experimental import pallas as pl
from jax.experimental.pallas import tpu as pltpu
from jax.experimental.pallas import tpu_sc as plsc

G = 64
EK = 125


def _init_body(x_ref, wT_ref, b_ref, wg_ref, h_ref, m_ref):
    h = jnp.dot(x_ref[...], wT_ref[...], preferred_element_type=jnp.float32)
    h = h + b_ref[...]
    h_ref[...] = h
    m = jnp.dot(h, wg_ref[...], preferred_element_type=jnp.float32)
    m_ref[...] = m.astype(m_ref.dtype)


def _init_proj(x, lin_WT, lin_b2, wg0, blk):
    n, f = x.shape
    return pl.pallas_call(
        _init_body,
        grid=(n // blk,),
        in_specs=[
            pl.BlockSpec((blk, f), lambda i: (i, 0)),
            pl.BlockSpec((f, f), lambda i: (0, 0)),
            pl.BlockSpec((1, f), lambda i: (0, 0)),
            pl.BlockSpec((f, f), lambda i: (0, 0)),
        ],
        out_specs=[
            pl.BlockSpec((blk, f), lambda i: (i, 0)),
            pl.BlockSpec((blk, f), lambda i: (i, 0)),
        ],
        out_shape=[jax.ShapeDtypeStruct((n, f), jnp.float32)] * 2,
    )(x, lin_WT, lin_b2, wg0)


def _gru_body(h_ref, agg_ref, wihT_ref, whhT_ref, bih_ref, bhh_ref, wgn_ref,
              hn_ref, mn_ref):
    f = h_ref.shape[1]
    h = h_ref[...]
    agg = agg_ref[0] + agg_ref[1]
    gi = jnp.dot(agg, wihT_ref[...], preferred_element_type=jnp.float32)
    gi = gi + bih_ref[...]
    gh = jnp.dot(h, whhT_ref[...], preferred_element_type=jnp.float32)
    gh = gh + bhh_ref[...]
    r = jax.nn.sigmoid(gi[:, :f] + gh[:, :f])
    z = jax.nn.sigmoid(gi[:, f:2 * f] + gh[:, f:2 * f])
    n = jnp.tanh(gi[:, 2 * f:] + r * gh[:, 2 * f:])
    hn = (1.0 - z) * n + z * h
    hn_ref[...] = hn
    mn = jnp.dot(hn, wgn_ref[...], preferred_element_type=jnp.float32)
    mn_ref[...] = mn.astype(mn_ref.dtype)


def _gru_step(h, agg2, wihT, whhT, bih2, bhh2, wg_next, blk):
    n, f = h.shape
    return pl.pallas_call(
        _gru_body,
        grid=(n // blk,),
        in_specs=[
            pl.BlockSpec((blk, f), lambda i: (i, 0)),
            pl.BlockSpec((2, blk, f), lambda i: (0, i, 0)),
            pl.BlockSpec((f, 3 * f), lambda i: (0, 0)),
            pl.BlockSpec((f, 3 * f), lambda i: (0, 0)),
            pl.BlockSpec((1, 3 * f), lambda i: (0, 0)),
            pl.BlockSpec((1, 3 * f), lambda i: (0, 0)),
            pl.BlockSpec((f, f), lambda i: (0, 0)),
        ],
        out_specs=[
            pl.BlockSpec((blk, f), lambda i: (i, 0)),
            pl.BlockSpec((blk, f), lambda i: (i, 0)),
        ],
        out_shape=[jax.ShapeDtypeStruct((n, f), jnp.float32)] * 2,
    )(h, agg2, wihT, whhT, bih2, bhh2, wg_next)


def _final_body(h_ref, agg_ref, wihT_ref, whhT_ref, bih_ref, bhh_ref,
                b3_ref, c1T_ref, c1b_ref, c2T_ref, c2b_ref, out_ref,
                sum_ref, cnt_ref):
    i = pl.program_id(0)
    nb = pl.num_programs(0)
    g = sum_ref.shape[0]
    f = h_ref.shape[1]

    @pl.when(i == 0)
    def _():
        sum_ref[...] = jnp.zeros_like(sum_ref)
        cnt_ref[...] = jnp.zeros_like(cnt_ref)

    h = h_ref[...]
    agg = agg_ref[0] + agg_ref[1]
    gi = jnp.dot(agg, wihT_ref[...], preferred_element_type=jnp.float32)
    gi = gi + bih_ref[...]
    gh = jnp.dot(h, whhT_ref[...], preferred_element_type=jnp.float32)
    gh = gh + bhh_ref[...]
    r = jax.nn.sigmoid(gi[:, :f] + gh[:, :f])
    z = jax.nn.sigmoid(gi[:, f:2 * f] + gh[:, f:2 * f])
    n = jnp.tanh(gi[:, 2 * f:] + r * gh[:, 2 * f:])
    hn = (1.0 - z) * n + z * h

    batch = b3_ref[0, 0, :]
    gids = lax.broadcasted_iota(jnp.int32, (g, batch.shape[0]), 0)
    onehot = (batch[None, :] == gids).astype(jnp.float32)
    sum_ref[...] += jnp.dot(onehot, hn, preferred_element_type=jnp.float32)
    cnt_ref[...] += jnp.sum(onehot, axis=1, keepdims=True)

    @pl.when(i == nb - 1)
    def _():
        pooled = sum_ref[...] / jnp.maximum(cnt_ref[...], 1.0)
        hid = jnp.dot(pooled, c1T_ref[...], preferred_element_type=jnp.float32)
        hid = jax.nn.relu(hid + c1b_ref[...])
        logits = jnp.dot(hid, c2T_ref[...], preferred_element_type=jnp.float32)
        out_ref[...] = jax.nn.sigmoid(logits + c2b_ref[...])


def _final_step(h, agg2, wihT, whhT, bih2, bhh2, batch3, c1T, c1b2, c2T,
                c2b2, blk):
    n, f = h.shape
    nb = n // blk
    return pl.pallas_call(
        _final_body,
        grid=(nb,),
        in_specs=[
            pl.BlockSpec((blk, f), lambda i: (i, 0)),
            pl.BlockSpec((2, blk, f), lambda i: (0, i, 0)),
            pl.BlockSpec((f, 3 * f), lambda i: (0, 0)),
            pl.BlockSpec((f, 3 * f), lambda i: (0, 0)),
            pl.BlockSpec((1, 3 * f), lambda i: (0, 0)),
            pl.BlockSpec((1, 3 * f), lambda i: (0, 0)),
            pl.BlockSpec((1, 1, blk), lambda i: (i, 0, 0)),
            pl.BlockSpec((f, f), lambda i: (0, 0)),
            pl.BlockSpec((1, f), lambda i: (0, 0)),
            pl.BlockSpec((f, 1), lambda i: (0, 0)),
            pl.BlockSpec((1, 1), lambda i: (0, 0)),
        ],
        out_specs=pl.BlockSpec((G, 1), lambda i: (0, 0)),
        out_shape=jax.ShapeDtypeStruct((G, 1), jnp.float32),
        scratch_shapes=[
            pltpu.VMEM((G, f), jnp.float32),
            pltpu.VMEM((G, 1), jnp.float32),
        ],
    )(h, agg2, wihT, whhT, bih2, bhh2, batch3, c1T, c1b2, c2T, c2b2)


@functools.lru_cache(maxsize=None)
def _make_edge_agg(n, f, nchunk):
    info = plsc.get_sparse_core_info()
    nc, ns = info.num_cores, info.num_subcores
    nw = nc * ns
    cpw = nchunk // nw
    rpt = (n // ns) // 8 * 8
    tail = n - rpt * ns
    mesh = plsc.VectorSubcoreMesh(core_axis_name="c", subcore_axis_name="s")

    nbuf = 2
    nph = 2
    hpc = cpw // nph
    nq = hpc // nbuf

    @functools.partial(
        pl.kernel, mesh=mesh,
        out_type=jax.ShapeDtypeStruct((nc, n, f), jnp.float32),
        scratch_types=[
            pltpu.VMEM((hpc, EK), jnp.int32),
            pltpu.VMEM((hpc, EK), jnp.int32),
        ] + [pltpu.VMEM((EK, f), jnp.float32)] * nbuf
          + [pltpu.VMEM_SHARED((n, f), jnp.float32)]
          + [pltpu.SemaphoreType.DMA] * nbuf,
    )
    def edge_agg(m_hbm, src_hbm, dst_hbm, z_hbm, out_hbm,
                 src_v, dst_v, *rest):
        bufs = rest[:nbuf]
        acc = rest[nbuf]
        gsems = rest[nbuf + 1:nbuf + 1 + nbuf]
        c = lax.axis_index("c")
        s = lax.axis_index("s")
        wid = s * nc + c
        base_r = s * rpt

        def gather(j, b):
            pltpu.async_copy(m_hbm.at[src_v.at[j]], bufs[b], gsems[b])

        def gather_wait(j, b):
            pltpu.make_async_copy(m_hbm.at[src_v.at[j]], bufs[b],
                                  gsems[b]).wait()

        def scat(j, b):
            pltpu.sync_copy(bufs[b], acc.at[dst_v.at[j]], add=True)

        def body(q, carry, last):
            j0 = q * nbuf
            for b in range(nbuf):
                gather_wait(j0 + b, b)
                scat(j0 + b, b)
                if not last:
                    gather(j0 + nbuf + b, b)
            return carry

        pltpu.sync_copy(src_hbm.at[pl.ds(wid * cpw, hpc)], src_v)
        pltpu.sync_copy(dst_hbm.at[pl.ds(wid * cpw, hpc)], dst_v)
        for b in range(nbuf):
            gather(b, b)
        pltpu.sync_copy(z_hbm.at[pl.ds(base_r, rpt)], acc.at[pl.ds(base_r, rpt)])
        if tail:
            @pl.when(s == ns - 1)
            def _():
                pltpu.sync_copy(z_hbm.at[pl.ds(rpt * ns, tail)],
                                acc.at[pl.ds(rpt * ns, tail)])
        plsc.subcore_barrier()

        for ph in range(nph):
            if ph:
                cbase = wid * cpw + ph * hpc
                pltpu.sync_copy(src_hbm.at[pl.ds(cbase, hpc)], src_v)
                pltpu.sync_copy(dst_hbm.at[pl.ds(cbase, hpc)], dst_v)
                for b in range(nbuf):
                    gather(b, b)
            lax.fori_loop(0, nq - 1, lambda q, cy: body(q, cy, False), 0)
            body(nq - 1, 0, True)
        plsc.subcore_barrier()
        pltpu.sync_copy(acc.at[pl.ds(base_r, rpt)],
                        out_hbm.at[c].at[pl.ds(base_r, rpt)])
        if tail:
            @pl.when(s == ns - 1)
            def _():
                pltpu.sync_copy(acc.at[pl.ds(rpt * ns, tail)],
                                out_hbm.at[c].at[pl.ds(rpt * ns, tail)])

    return edge_agg


def kernel(x, edge_index, batch, lin_W, lin_b, ggnn_W, gru_W_ih, gru_W_hh,
           gru_b_ih, gru_b_hh, c1_W, c1_b, c2_W, c2_b):
    n, f = x.shape
    num_layers = ggnn_W.shape[0]
    e = edge_index.shape[1]
    blk = 5000

    src2d = edge_index[0].reshape(e // EK, EK)
    dst2d = edge_index[1].reshape(e // EK, EK)
    zeros = jnp.zeros((n, f), jnp.float32)
    lin_WT = lin_W.T
    lin_b2 = lin_b.reshape(1, f)
    wihT = gru_W_ih.T
    whhT = gru_W_hh.T
    bih2 = gru_b_ih.reshape(1, 3 * f)
    bhh2 = gru_b_hh.reshape(1, 3 * f)
    c1T = c1_W.T
    c1b2 = c1_b.reshape(1, f)
    c2T = c2_W.T
    c2b2 = c2_b.reshape(1, 1)
    batch3 = batch.reshape(n // blk, 1, blk)

    edge_agg = _make_edge_agg(n, f, e // EK)

    h, m = _init_proj(x, lin_WT, lin_b2, ggnn_W[0], blk)
    for i in range(num_layers - 1):
        agg2 = edge_agg(m, src2d, dst2d, zeros)
        h, m = _gru_step(h, agg2, wihT, whhT, bih2, bhh2, ggnn_W[i + 1], blk)
    agg2 = edge_agg(m, src2d, dst2d, zeros)
    return _final_step(h, agg2, wihT, whhT, bih2, bhh2, batch3, c1T, c1b2,
                       c2T, c2b2, blk)
